# baseline (speedup 1.0000x reference)
import functools

import jax
import jax.numpy as jnp
from jax import lax
from jax.experimental import pallas as pl
from jax.experimental.pallas import tpu as pltpu
from jax.experimental.pallas import tpu_sc as plsc

N_DOCS = 8
DOC_LEN = 256
M = DOC_LEN - 1
K = M - 1
TOTAL = N_DOCS * DOC_LEN


def _shifts_right(v, n):
    a = v[None, :]
    step = 1
    while a.shape[0] < n:
        z = jnp.zeros((a.shape[0], step), jnp.float32)
        shifted = jnp.concatenate([z, a[:, : M - step]], axis=1)
        a = jnp.concatenate([a, shifted], axis=0)
        step *= 2
    return a


def _shifts_left(v, n):
    a = v[None, :]
    step = 1
    while a.shape[0] < n:
        z = jnp.zeros((a.shape[0], step), jnp.float32)
        shifted = jnp.concatenate([a[:, step:], z], axis=1)
        a = jnp.concatenate([a, shifted], axis=0)
        step *= 2
    return a


def _gate_tc_body(s_ref, fwd_out_ref, bwd_out_ref):
    s = s_ref[0, 0, :]
    fwd = s[:M]
    bwd = s[1:]

    u = (lax.broadcasted_iota(jnp.int32, (K, K), 0)
         <= lax.broadcasted_iota(jnp.int32, (K, K), 1)).astype(jnp.float32)

    def gate(t, base):
        x = (t - base[None, :]) * 100.0 + 5.0
        lsg = jnp.minimum(x, 0.0) - jnp.log1p(jnp.exp(-jnp.abs(x)))
        c = lax.dot_general(lsg, u, (((0,), (0,)), ((), ())),
                            preferred_element_type=jnp.float32,
                            precision=lax.Precision.HIGHEST)
        return jnp.exp(c)

    t_fwd = _shifts_right(fwd, DOC_LEN)[:K]
    bwd1 = jnp.concatenate([bwd[1:], jnp.zeros((1,), jnp.float32)])
    t_bwd = _shifts_left(bwd1, DOC_LEN)[:K]

    fwd_out_ref[0] = gate(t_fwd, fwd)
    bwd_out_ref[0] = gate(t_bwd, bwd)


def _gate_tc(s3d, interpret=False):
    return pl.pallas_call(
        _gate_tc_body,
        grid=(N_DOCS,),
        in_specs=[pl.BlockSpec((1, 1, DOC_LEN), lambda d: (d, 0, 0))],
        out_specs=[pl.BlockSpec((1, M, K), lambda d: (d, 0, 0)),
                   pl.BlockSpec((1, M, K), lambda d: (d, 0, 0))],
        out_shape=[jax.ShapeDtypeStruct((N_DOCS, M, K), jnp.float32),
                   jax.ShapeDtypeStruct((N_DOCS, M, K), jnp.float32)],
        interpret=interpret,
    )(s3d)


def _sc_gather(score, rep_idx):
    info = plsc.get_sparse_core_info()
    nw = info.num_cores * info.num_subcores
    chunk = TOTAL // nw
    per_doc = nw // N_DOCS
    mesh = plsc.VectorSubcoreMesh(core_axis_name="c", subcore_axis_name="s")

    @functools.partial(
        pl.kernel, mesh=mesh,
        out_type=jax.ShapeDtypeStruct((N_DOCS, 1, DOC_LEN), jnp.float32),
        compiler_params=pltpu.CompilerParams(needs_layout_passes=False),
        scratch_types=[pltpu.VMEM((TOTAL,), jnp.float32),
                       pltpu.VMEM((chunk,), jnp.int32),
                       pltpu.VMEM((chunk,), jnp.float32)],
    )
    def k(score_hbm, idx_hbm, out_hbm, score_v, idx_v, vals_v):
        wid = lax.axis_index("s") * info.num_cores + lax.axis_index("c")
        doc = wid // per_doc
        q = wid % per_doc
        pltpu.sync_copy(score_hbm, score_v)
        pltpu.sync_copy(idx_hbm.at[doc, pl.ds(q * chunk, chunk)], idx_v)
        for r in range(chunk // 16):
            idx = idx_v[pl.ds(r * 16, 16)]
            vals_v[pl.ds(r * 16, 16)] = plsc.load_gather(score_v, [idx])
        pltpu.sync_copy(vals_v, out_hbm.at[doc, 0, pl.ds(q * chunk, chunk)])

    return k(score, rep_idx)


def kernel(score, rep_srcs, rep_idx):
    del rep_srcs
    s3d = _sc_gather(score, rep_idx)
    return tuple(_gate_tc(s3d))


# speedup vs baseline: 2.1987x; 2.1987x over previous
import functools

import jax
import jax.numpy as jnp
from jax import lax
from jax.experimental import pallas as pl
from jax.experimental.pallas import tpu as pltpu
from jax.experimental.pallas import tpu_sc as plsc

N_DOCS = 8
DOC_LEN = 256
M = DOC_LEN - 1
K = M - 1
TOTAL = N_DOCS * DOC_LEN


def _shifts_right(v, n):
    a = v[None, :]
    step = 1
    while a.shape[0] < n:
        z = jnp.zeros((a.shape[0], step), jnp.float32)
        shifted = jnp.concatenate([z, a[:, : M - step]], axis=1)
        a = jnp.concatenate([a, shifted], axis=0)
        step *= 2
    return a


def _shifts_left(v, n):
    a = v[None, :]
    step = 1
    while a.shape[0] < n:
        z = jnp.zeros((a.shape[0], step), jnp.float32)
        shifted = jnp.concatenate([a[:, step:], z], axis=1)
        a = jnp.concatenate([a, shifted], axis=0)
        step *= 2
    return a


def _gate_tc_body(s_ref, fwd_out_ref, bwd_out_ref):
    s = s_ref[0, 0, :]
    fwd = s[:M]
    bwd = s[1:]

    u = (lax.broadcasted_iota(jnp.int32, (K, K), 0)
         <= lax.broadcasted_iota(jnp.int32, (K, K), 1)).astype(jnp.float32)

    def gate(t, base):
        x = (t - base[None, :]) * 100.0 + 5.0
        lsg = jnp.minimum(x, 0.0) - jnp.log1p(jnp.exp(-jnp.abs(x)))
        c = lax.dot_general(lsg, u, (((0,), (0,)), ((), ())),
                            preferred_element_type=jnp.float32,
                            precision=lax.Precision.HIGHEST)
        return jnp.exp(c)

    t_fwd = _shifts_right(fwd, DOC_LEN)[:K]
    bwd1 = jnp.concatenate([bwd[1:], jnp.zeros((1,), jnp.float32)])
    t_bwd = _shifts_left(bwd1, DOC_LEN)[:K]

    fwd_out_ref[0] = gate(t_fwd, fwd)
    bwd_out_ref[0] = gate(t_bwd, bwd)


def _gate_tc(s3d, interpret=False):
    return pl.pallas_call(
        _gate_tc_body,
        grid=(N_DOCS,),
        in_specs=[pl.BlockSpec((1, 1, DOC_LEN), lambda d: (d, 0, 0))],
        out_specs=[pl.BlockSpec((1, M, K), lambda d: (d, 0, 0)),
                   pl.BlockSpec((1, M, K), lambda d: (d, 0, 0))],
        out_shape=[jax.ShapeDtypeStruct((N_DOCS, M, K), jnp.float32),
                   jax.ShapeDtypeStruct((N_DOCS, M, K), jnp.float32)],
        interpret=interpret,
    )(s3d)


def _sc_gather(score, rep_idx):
    info = plsc.get_sparse_core_info()
    nw = info.num_cores * info.num_subcores
    chunk = TOTAL // nw
    per_doc = nw // N_DOCS
    mesh = plsc.VectorSubcoreMesh(core_axis_name="c", subcore_axis_name="s")

    @functools.partial(
        pl.kernel, mesh=mesh,
        out_type=jax.ShapeDtypeStruct((N_DOCS, 1, DOC_LEN), jnp.float32),
        compiler_params=pltpu.CompilerParams(needs_layout_passes=False),
        scratch_types=[pltpu.VMEM((TOTAL,), jnp.float32),
                       pltpu.VMEM((chunk,), jnp.int32),
                       pltpu.VMEM((chunk,), jnp.float32)],
    )
    def k(score_hbm, idx_hbm, out_hbm, score_v, idx_v, vals_v):
        wid = lax.axis_index("s") * info.num_cores + lax.axis_index("c")
        doc = wid // per_doc
        q = wid % per_doc
        pltpu.sync_copy(score_hbm, score_v)
        pltpu.sync_copy(idx_hbm.at[doc, pl.ds(q * chunk, chunk)], idx_v)
        for r in range(chunk // 16):
            idx = idx_v[pl.ds(r * 16, 16)]
            vals_v[pl.ds(r * 16, 16)] = plsc.load_gather(score_v, [idx])
        pltpu.sync_copy(vals_v, out_hbm.at[doc, 0, pl.ds(q * chunk, chunk)])

    return k(score, rep_idx)


def kernel(score, rep_srcs, rep_idx):
    del rep_srcs
    del rep_idx
    s3d = score.reshape(N_DOCS, 1, DOC_LEN)
    return tuple(_gate_tc(s3d))
